# Initial kernel scaffold; baseline (speedup 1.0000x reference)
import jax
import jax.numpy as jnp
from jax.experimental import pallas as pl


def kernel(boxes, scores):
    raise NotImplementedError("write your pallas kernel here")


# speedup vs baseline: 349.1523x; 349.1523x over previous
import jax
import jax.numpy as jnp
from jax.experimental import pallas as pl
from jax.experimental.pallas import tpu as pltpu

_N = 20000
_PRE = 6000
_POST = 300
_TH = 0.7
_T = 128
_NT = 48
_NPAD = _T * _NT


def _nms_body(ct_ref, out_ref, sup_ref):
    x1 = ct_ref[0:1, :]
    y1 = ct_ref[1:2, :]
    x2 = ct_ref[2:3, :]
    y2 = ct_ref[3:4, :]
    areas = (x2 - x1) * (y2 - y1)

    sup_ref[...] = jnp.zeros((1, _NPAD), jnp.float32)

    def tile_step(t, _):
        s = t * _T
        tile = ct_ref[:, pl.ds(s, _T)]
        tile_t = jnp.transpose(tile)
        tx1 = tile_t[:, 0:1]
        ty1 = tile_t[:, 1:2]
        tx2 = tile_t[:, 2:3]
        ty2 = tile_t[:, 3:4]

        xx1 = jnp.maximum(tx1, x1)
        yy1 = jnp.maximum(ty1, y1)
        xx2 = jnp.minimum(tx2, x2)
        yy2 = jnp.minimum(ty2, y2)
        inter = jnp.maximum(xx2 - xx1, 0.0) * jnp.maximum(yy2 - yy1, 0.0)
        t_area = (tx2 - tx1) * (ty2 - ty1)
        iou = inter / (t_area + areas - inter + 1e-9)
        m = jnp.where(iou > _TH, 1.0, 0.0)

        rx1 = tile[0:1, :]
        ry1 = tile[1:2, :]
        rx2 = tile[2:3, :]
        ry2 = tile[3:4, :]
        sxx1 = jnp.maximum(tx1, rx1)
        syy1 = jnp.maximum(ty1, ry1)
        sxx2 = jnp.minimum(tx2, rx2)
        syy2 = jnp.minimum(ty2, ry2)
        inter_s = jnp.maximum(sxx2 - sxx1, 0.0) * jnp.maximum(syy2 - syy1, 0.0)
        r_area = (rx2 - rx1) * (ry2 - ry1)
        iou_s = inter_s / (t_area + r_area - inter_s + 1e-9)
        row_i = jax.lax.broadcasted_iota(jnp.int32, (_T, _T), 0)
        col_j = jax.lax.broadcasted_iota(jnp.int32, (_T, _T), 1)
        s_mat = jnp.where((iou_s > _TH) & (row_i < col_j), 1.0, 0.0)

        alive = 1.0 - sup_ref[0:1, pl.ds(s, _T)]

        def cond(carry):
            return carry[1]

        def body(carry):
            k, _ = carry
            hit = jax.lax.dot_general(
                k, s_mat, (((1,), (0,)), ((), ())),
                preferred_element_type=jnp.float32)
            k_new = jnp.where(hit > 0.5, 0.0, alive)
            return k_new, jnp.any(k_new != k)

        k, _ = jax.lax.while_loop(cond, body, (alive, True))

        sc_tile = ct_ref[4:5, pl.ds(s, _T)]
        out_ref[0:1, pl.ds(s, _T)] = jnp.where(k > 0.5, sc_tile, -1e9)

        hit_all = jax.lax.dot_general(
            k, m, (((1,), (0,)), ((), ())),
            preferred_element_type=jnp.float32)
        sup_ref[...] = jnp.maximum(sup_ref[...], jnp.where(hit_all > 0.5, 1.0, 0.0))
        return 0

    jax.lax.fori_loop(0, _NT, tile_step, 0)


def _masked_scores(top_boxes, top_scores):
    ct = jnp.zeros((8, _NPAD), jnp.float32)
    ct = ct.at[0:4, :_PRE].set(top_boxes.T)
    ct = ct.at[4, :_PRE].set(top_scores)
    out = pl.pallas_call(
        _nms_body,
        out_shape=jax.ShapeDtypeStruct((1, _NPAD), jnp.float32),
        scratch_shapes=[pltpu.VMEM((1, _NPAD), jnp.float32)],
    )(ct)
    return out[0, :_PRE]


def kernel(boxes, scores):
    obj = jax.nn.softmax(scores, axis=1)[:, 1]
    top_scores, idx = jax.lax.top_k(obj, _PRE)
    top_boxes = jnp.take(boxes, idx, axis=0)
    masked = _masked_scores(top_boxes, top_scores)
    sel_scores, sel_idx = jax.lax.top_k(masked, _POST)
    sel_boxes = jnp.take(top_boxes, sel_idx, axis=0)
    return jnp.concatenate([sel_boxes, sel_scores[:, None]], axis=1)
